# Initial kernel scaffold; baseline (speedup 1.0000x reference)
import jax
import jax.numpy as jnp
from jax.experimental import pallas as pl


def kernel(num_0, num_1, num_2, num_3, num_4, num_5, num_6, num_7, num_8, num_9, num_10, num_11, num_12, cat_0, cat_1, cat_2, cat_3, cat_4, cat_5, cat_6, cat_7, cat_8, cat_9, cat_10, cat_11, cat_12, cat_13, cat_14, cat_15, cat_16, cat_17, cat_18, cat_19, cat_20, cat_21, cat_22, cat_23, cat_24, cat_25, W_0, W_1, W_2, W_3, W_4, W_5, W_6, W_7, W_8, W_9, W_10, W_11, W_12, W_13, W_14, W_15, W_16, W_17, W_18, W_19, W_20, W_21, W_22, W_23, W_24, W_25):
    raise NotImplementedError("write your pallas kernel here")


# speedup vs baseline: 7.1116x; 7.1116x over previous
import functools

import jax
import jax.numpy as jnp
import numpy as np
from jax.experimental import pallas as pl
from jax.experimental.pallas import tpu as pltpu
from jax.experimental.pallas import tpu_sc as plsc

_B = 16384
_EMB = 32
_NUMC = 13
_VOCABS = (1000,) * 6 + (100,) * 20
_NF = len(_VOCABS)
_NIDX = _NF * _B
_OFFSETS = np.concatenate([[0], np.cumsum(_VOCABS)[:-1]]).astype(np.int32)

_WINDOW = 128
_TBLK = 2048


def _cats_gather(table, idx2d):
    mesh = plsc.VectorSubcoreMesh(core_axis_name="c", subcore_axis_name="s")

    @functools.partial(
        pl.kernel,
        out_type=jax.ShapeDtypeStruct((_NIDX, _EMB), jnp.float32),
        mesh=mesh,
        compiler_params=pltpu.CompilerParams(use_tc_tiling_on_sc=False),
    )
    def k(tbl_hbm, idx_hbm, out_hbm):
        def body(i_vmem, o_vmem):
            pltpu.sync_copy(tbl_hbm.at[i_vmem.at[0]], o_vmem)

        pltpu.emit_pipeline(
            body,
            grid=(_NIDX // _WINDOW,),
            in_specs=[pl.BlockSpec((1, _WINDOW), lambda i: (0, i))],
            out_specs=[pl.BlockSpec((_WINDOW, _EMB), lambda i: (i, 0))],
            core_axis_name=("c", "s"),
            dimension_semantics=(pltpu.PARALLEL,),
        )(idx_hbm, out_hbm)

    return k(table, idx2d)


def _nums_transpose(stacked):
    def body(in_ref, out_ref):
        out_ref[...] = in_ref[...].T

    return pl.pallas_call(
        body,
        grid=(_B // _TBLK,),
        in_specs=[pl.BlockSpec((_NUMC, _TBLK), lambda j: (0, j))],
        out_specs=pl.BlockSpec((_TBLK, _NUMC), lambda j: (j, 0)),
        out_shape=jax.ShapeDtypeStruct((_B, _NUMC), jnp.float32),
    )(stacked)


def kernel(num_0, num_1, num_2, num_3, num_4, num_5, num_6, num_7, num_8,
           num_9, num_10, num_11, num_12,
           cat_0, cat_1, cat_2, cat_3, cat_4, cat_5, cat_6, cat_7, cat_8,
           cat_9, cat_10, cat_11, cat_12, cat_13, cat_14, cat_15, cat_16,
           cat_17, cat_18, cat_19, cat_20, cat_21, cat_22, cat_23, cat_24,
           cat_25,
           W_0, W_1, W_2, W_3, W_4, W_5, W_6, W_7, W_8, W_9, W_10, W_11,
           W_12, W_13, W_14, W_15, W_16, W_17, W_18, W_19, W_20, W_21,
           W_22, W_23, W_24, W_25):
    nums = [num_0, num_1, num_2, num_3, num_4, num_5, num_6, num_7, num_8,
            num_9, num_10, num_11, num_12]
    cats = [cat_0, cat_1, cat_2, cat_3, cat_4, cat_5, cat_6, cat_7, cat_8,
            cat_9, cat_10, cat_11, cat_12, cat_13, cat_14, cat_15, cat_16,
            cat_17, cat_18, cat_19, cat_20, cat_21, cat_22, cat_23, cat_24,
            cat_25]
    tables = [W_0, W_1, W_2, W_3, W_4, W_5, W_6, W_7, W_8, W_9, W_10, W_11,
              W_12, W_13, W_14, W_15, W_16, W_17, W_18, W_19, W_20, W_21,
              W_22, W_23, W_24, W_25]

    table = jnp.concatenate(tables, axis=0)
    idx = (jnp.stack(cats, axis=1) + jnp.asarray(_OFFSETS)[None, :])
    idx2d = idx.reshape(1, _NIDX)

    gathered = _cats_gather(table, idx2d)
    x_cats = gathered.reshape(_B, _NF * _EMB)

    x_nums = _nums_transpose(jnp.stack(nums, axis=0))
    return (x_nums, x_cats)


# speedup vs baseline: 7.8987x; 1.1107x over previous
import functools

import jax
import jax.numpy as jnp
import numpy as np
from jax.experimental import pallas as pl
from jax.experimental.pallas import tpu as pltpu
from jax.experimental.pallas import tpu_sc as plsc

_B = 16384
_EMB = 32
_NUMC = 13
_VOCABS = (1000,) * 6 + (100,) * 20
_NF = len(_VOCABS)
_NIDX = _NF * _B
_OFFSETS = np.concatenate([[0], np.cumsum(_VOCABS)[:-1]]).astype(np.int32)

_WINDOW = 256
_TBLK = 2048


def _cats_gather(table, idx2d):
    mesh = plsc.VectorSubcoreMesh(core_axis_name="c", subcore_axis_name="s")

    @functools.partial(
        pl.kernel,
        out_type=jax.ShapeDtypeStruct((_NIDX, _EMB), jnp.float32),
        mesh=mesh,
        compiler_params=pltpu.CompilerParams(use_tc_tiling_on_sc=False),
    )
    def k(tbl_hbm, idx_hbm, out_hbm):
        def body(i_vmem, o_vmem):
            pltpu.sync_copy(tbl_hbm.at[i_vmem.at[0]], o_vmem)

        pltpu.emit_pipeline(
            body,
            grid=(_NIDX // _WINDOW,),
            in_specs=[pl.BlockSpec((1, _WINDOW), lambda i: (0, i))],
            out_specs=[pl.BlockSpec((_WINDOW, _EMB), lambda i: (i, 0))],
            core_axis_name=("c", "s"),
            dimension_semantics=(pltpu.PARALLEL,),
        )(idx_hbm, out_hbm)

    return k(table, idx2d)


def _nums_transpose(stacked):
    def body(in_ref, out_ref):
        out_ref[...] = in_ref[...].T

    return pl.pallas_call(
        body,
        grid=(_B // _TBLK,),
        in_specs=[pl.BlockSpec((_NUMC, _TBLK), lambda j: (0, j))],
        out_specs=pl.BlockSpec((_TBLK, _NUMC), lambda j: (j, 0)),
        out_shape=jax.ShapeDtypeStruct((_B, _NUMC), jnp.float32),
    )(stacked)


def kernel(num_0, num_1, num_2, num_3, num_4, num_5, num_6, num_7, num_8,
           num_9, num_10, num_11, num_12,
           cat_0, cat_1, cat_2, cat_3, cat_4, cat_5, cat_6, cat_7, cat_8,
           cat_9, cat_10, cat_11, cat_12, cat_13, cat_14, cat_15, cat_16,
           cat_17, cat_18, cat_19, cat_20, cat_21, cat_22, cat_23, cat_24,
           cat_25,
           W_0, W_1, W_2, W_3, W_4, W_5, W_6, W_7, W_8, W_9, W_10, W_11,
           W_12, W_13, W_14, W_15, W_16, W_17, W_18, W_19, W_20, W_21,
           W_22, W_23, W_24, W_25):
    nums = [num_0, num_1, num_2, num_3, num_4, num_5, num_6, num_7, num_8,
            num_9, num_10, num_11, num_12]
    cats = [cat_0, cat_1, cat_2, cat_3, cat_4, cat_5, cat_6, cat_7, cat_8,
            cat_9, cat_10, cat_11, cat_12, cat_13, cat_14, cat_15, cat_16,
            cat_17, cat_18, cat_19, cat_20, cat_21, cat_22, cat_23, cat_24,
            cat_25]
    tables = [W_0, W_1, W_2, W_3, W_4, W_5, W_6, W_7, W_8, W_9, W_10, W_11,
              W_12, W_13, W_14, W_15, W_16, W_17, W_18, W_19, W_20, W_21,
              W_22, W_23, W_24, W_25]

    table = jnp.concatenate(tables, axis=0)
    idx = (jnp.stack(cats, axis=1) + jnp.asarray(_OFFSETS)[None, :])
    idx2d = idx.reshape(1, _NIDX)

    gathered = _cats_gather(table, idx2d)
    x_cats = gathered.reshape(_B, _NF * _EMB)

    x_nums = _nums_transpose(jnp.stack(nums, axis=0))
    return (x_nums, x_cats)
